# Initial kernel scaffold; baseline (speedup 1.0000x reference)
import jax
import jax.numpy as jnp
from jax.experimental import pallas as pl


def kernel(x, mask, pos_embed):
    raise NotImplementedError("write your pallas kernel here")


# speedup vs baseline: 1.0079x; 1.0079x over previous
import jax
import jax.numpy as jnp
from jax.experimental import pallas as pl


_BLOCK_S = 1024


def _add_pos_kernel(x_ref, mask_ref, pos_ref, out_ref):
    m = mask_ref[...]
    out_ref[0] = x_ref[0] + pos_ref[0] * m


def kernel(x, mask, pos_embed):
    B, S, D = x.shape
    maskf = mask.reshape(S, 1).astype(jnp.float32)
    bs = _BLOCK_S
    grid = (S // bs, B)
    return pl.pallas_call(
        _add_pos_kernel,
        grid=grid,
        in_specs=[
            pl.BlockSpec((1, bs, D), lambda i, j: (j, i, 0)),
            pl.BlockSpec((bs, 1), lambda i, j: (i, 0)),
            pl.BlockSpec((1, bs, D), lambda i, j: (0, i, 0)),
        ],
        out_specs=pl.BlockSpec((1, bs, D), lambda i, j: (j, i, 0)),
        out_shape=jax.ShapeDtypeStruct((B, S, D), x.dtype),
    )(x, maskf, pos_embed)


# speedup vs baseline: 1.0516x; 1.0434x over previous
import jax
import jax.numpy as jnp
from jax.experimental import pallas as pl


_BLOCK_S = 2048


def _add_pos_kernel(x_ref, mask_ref, pos_ref, out_ref):
    m = mask_ref[...]
    out_ref[0] = x_ref[0] + pos_ref[0] * m


def kernel(x, mask, pos_embed):
    B, S, D = x.shape
    maskf = mask.reshape(S, 1).astype(jnp.float32)
    bs = _BLOCK_S
    grid = (S // bs, B)
    return pl.pallas_call(
        _add_pos_kernel,
        grid=grid,
        in_specs=[
            pl.BlockSpec((1, bs, D), lambda i, j: (j, i, 0)),
            pl.BlockSpec((bs, 1), lambda i, j: (i, 0)),
            pl.BlockSpec((1, bs, D), lambda i, j: (0, i, 0)),
        ],
        out_specs=pl.BlockSpec((1, bs, D), lambda i, j: (j, i, 0)),
        out_shape=jax.ShapeDtypeStruct((B, S, D), x.dtype),
    )(x, maskf, pos_embed)
